# baseline (speedup 1.0000x reference)
import functools

import jax
import jax.numpy as jnp
from jax import lax
from jax.experimental import pallas as pl
from jax.experimental.pallas import tpu as pltpu
from jax.experimental.pallas import tpu_sc as plsc

BETA_P1 = 1.25

_NC, _NS, _L = 2, 16, 16


def _argmin_body(x_ref, cb_ref, idx_ref, loss_ref):
    b = pl.program_id(0)
    cb = cb_ref[...]
    C = cb_ref.shape[0]
    G = x_ref.shape[0]

    cb_sq = jnp.sum(cb * cb, axis=1, keepdims=True)
    iota_c = jax.lax.broadcasted_iota(jnp.int32, (C, 1), 0).astype(jnp.float32)

    @pl.when(b == 0)
    def _():
        loss_ref[0, 0] = 0.0

    for g in range(G):
        xb = x_ref[g]
        x_sq = jnp.sum(xb * xb, axis=0, keepdims=True)
        mm = jax.lax.dot_general(
            cb, xb, (((1,), (0,)), ((), ())),
            preferred_element_type=jnp.float32)
        dist = (0.5 * x_sq + 0.5 * cb_sq) - mm

        minv = jnp.min(dist, axis=0, keepdims=True)
        masked = jnp.where(dist == minv, iota_c, jnp.float32(C))
        idx = jnp.min(masked, axis=0, keepdims=True)
        idx_ref[g] = idx.astype(jnp.int32)
        loss_ref[0, 0] += jnp.sum(minv)


def _tc_argmin(x, codebook):
    B, D, HW = x.shape
    C = codebook.shape[0]
    G = 2
    idx, s = pl.pallas_call(
        _argmin_body,
        grid=(B // G,),
        in_specs=[
            pl.BlockSpec((G, D, HW), lambda b: (b, 0, 0)),
            pl.BlockSpec((C, D), lambda b: (0, 0)),
        ],
        out_specs=[
            pl.BlockSpec((G, 1, HW), lambda b: (b, 0, 0)),
            pl.BlockSpec(memory_space=pltpu.SMEM),
        ],
        out_shape=[
            jax.ShapeDtypeStruct((B, 1, HW), jnp.int32),
            jax.ShapeDtypeStruct((1, 1), jnp.float32),
        ],
    )(x, codebook)
    return idx, s


def _sc_lookup(cbt, idx_flat, B, D, HW):
    NW = _NC * _NS
    n_pos = B * HW
    p_per_w = n_pos // NW
    C = cbt.shape[1]
    mesh = plsc.VectorSubcoreMesh(core_axis_name="c", subcore_axis_name="s")

    @functools.partial(
        pl.kernel,
        mesh=mesh,
        compiler_params=pltpu.CompilerParams(needs_layout_passes=False),
        out_type=jax.ShapeDtypeStruct((B, D, HW), jnp.float32),
        scratch_types=[
            pltpu.VMEM((p_per_w,), jnp.int32),
            pltpu.VMEM((D, C), jnp.float32),
            pltpu.VMEM((D, p_per_w), jnp.float32),
        ],
    )
    def k(cbt_hbm, idx_hbm, out_hbm, idx_v, cbt_v, out_v):
        wid = lax.axis_index("s") * _NC + lax.axis_index("c")
        base = wid * p_per_w
        b_img = base // HW
        col0 = base - b_img * HW
        pltpu.sync_copy(idx_hbm.at[pl.ds(base, p_per_w)], idx_v)
        pltpu.sync_copy(cbt_hbm, cbt_v)

        @pl.loop(0, p_per_w, step=_L)
        def _(j):
            idx16 = idx_v[pl.ds(j, _L)]
            @pl.loop(0, D)
            def _(d):
                d16 = jnp.zeros((_L,), jnp.int32) + d
                row = plsc.load_gather(cbt_v, [d16, idx16])
                out_v[d, pl.ds(j, _L)] = row

        pltpu.sync_copy(out_v, out_hbm.at[b_img, :, pl.ds(col0, p_per_w)])

    return k(cbt, idx_flat)


def kernel(latents, codebook):
    B, D, H, W = latents.shape
    HW = H * W
    x = latents.reshape(B, D, HW)
    cbt = codebook.T

    idx, s = _tc_argmin(x, codebook)
    q = _sc_lookup(cbt, idx.reshape(B * HW), B, D, HW)

    vq_loss = (2.0 * BETA_P1 / (B * HW * D)) * s[0, 0]
    return (q.reshape(B, D, H, W), vq_loss)


# speedup vs baseline: 1.4643x; 1.4643x over previous
import jax
import jax.numpy as jnp
from jax.experimental import pallas as pl
from jax.experimental.pallas import tpu as pltpu

BETA_P1 = 1.25


def _tree_min(m):
    while m.shape[0] > 8:
        h = m.shape[0] // 2
        m = jnp.minimum(m[:h], m[h:])
    return jnp.min(m, axis=0, keepdims=True)


def _vq_body(x_ref, cb_ref, cbt_ref, q_ref, loss_ref):
    b = pl.program_id(0)
    cb = cb_ref[...]
    C = cb_ref.shape[0]
    G = x_ref.shape[0]

    cb_sq = jnp.sum(cb * cb, axis=1, keepdims=True)
    iota_c = jax.lax.broadcasted_iota(jnp.int32, (C, 1), 0).astype(jnp.float32)

    @pl.when(b == 0)
    def _():
        loss_ref[0, 0] = 0.0

    for g in range(G):
        xb = x_ref[g]
        x_sq = jnp.sum(xb * xb, axis=0, keepdims=True)
        mm = jax.lax.dot_general(
            cb, xb, (((1,), (0,)), ((), ())),
            preferred_element_type=jnp.float32)
        dist = (0.5 * x_sq + 0.5 * cb_sq) - mm

        minv = _tree_min(dist)
        masked = jnp.where(dist == minv, iota_c, jnp.float32(C))
        idx = _tree_min(masked)
        onehot = (iota_c == idx).astype(jnp.float32)

        q = jax.lax.dot_general(
            cbt_ref[...], onehot, (((1,), (0,)), ((), ())),
            preferred_element_type=jnp.float32)
        q_ref[g] = q
        loss_ref[0, 0] += jnp.sum(minv)


def kernel(latents, codebook):
    B, D, H, W = latents.shape
    C = codebook.shape[0]
    HW = H * W
    x = latents.reshape(B, D, HW)
    cbt = codebook.T

    G = 2
    q, s = pl.pallas_call(
        _vq_body,
        grid=(B // G,),
        in_specs=[
            pl.BlockSpec((G, D, HW), lambda b: (b, 0, 0)),
            pl.BlockSpec((C, D), lambda b: (0, 0)),
            pl.BlockSpec((D, C), lambda b: (0, 0)),
        ],
        out_specs=[
            pl.BlockSpec((G, D, HW), lambda b: (b, 0, 0)),
            pl.BlockSpec(memory_space=pltpu.SMEM),
        ],
        out_shape=[
            jax.ShapeDtypeStruct((B, D, HW), jnp.float32),
            jax.ShapeDtypeStruct((1, 1), jnp.float32),
        ],
    )(x, codebook, cbt)

    vq_loss = (2.0 * BETA_P1 / (B * HW * D)) * s[0, 0]
    return (q.reshape(B, D, H, W), vq_loss)


# speedup vs baseline: 1.5427x; 1.0536x over previous
import jax
import jax.numpy as jnp
from jax.experimental import pallas as pl
from jax.experimental.pallas import tpu as pltpu

BETA_P1 = 1.25


def _vq_body(x_ref, cb_ref, cbt_ref, q_ref, loss_ref):
    b = pl.program_id(0)
    cb = cb_ref[...]
    C = cb_ref.shape[0]
    G = x_ref.shape[0]

    cb_sq = jnp.sum(cb * cb, axis=1, keepdims=True)
    iota_c = jax.lax.broadcasted_iota(jnp.int32, (C, 1), 0).astype(jnp.float32)

    @pl.when(b == 0)
    def _():
        loss_ref[0, 0] = 0.0

    for g in range(G):
        xb = x_ref[g]
        x_sq = jnp.sum(xb * xb, axis=0, keepdims=True)
        mm = jax.lax.dot_general(
            cb, xb, (((1,), (0,)), ((), ())),
            preferred_element_type=jnp.float32)
        dist = (0.5 * x_sq + 0.5 * cb_sq) - mm

        minv = jnp.min(dist, axis=0, keepdims=True)
        masked = jnp.where(dist == minv, iota_c, jnp.float32(C))
        idx = jnp.min(masked, axis=0, keepdims=True)
        onehot = (iota_c == idx).astype(jnp.float32)

        q = jax.lax.dot_general(
            cbt_ref[...], onehot, (((1,), (0,)), ((), ())),
            preferred_element_type=jnp.float32)
        q_ref[g] = q
        loss_ref[0, 0] += jnp.sum(minv)


def kernel(latents, codebook):
    B, D, H, W = latents.shape
    C = codebook.shape[0]
    HW = H * W
    x = latents.reshape(B, D, HW)
    cbt = codebook.T

    G = 4
    q, s = pl.pallas_call(
        _vq_body,
        grid=(B // G,),
        in_specs=[
            pl.BlockSpec((G, D, HW), lambda b: (b, 0, 0)),
            pl.BlockSpec((C, D), lambda b: (0, 0)),
            pl.BlockSpec((D, C), lambda b: (0, 0)),
        ],
        out_specs=[
            pl.BlockSpec((G, D, HW), lambda b: (b, 0, 0)),
            pl.BlockSpec(memory_space=pltpu.SMEM),
        ],
        out_shape=[
            jax.ShapeDtypeStruct((B, D, HW), jnp.float32),
            jax.ShapeDtypeStruct((1, 1), jnp.float32),
        ],
    )(x, codebook, cbt)

    vq_loss = (2.0 * BETA_P1 / (B * HW * D)) * s[0, 0]
    return (q.reshape(B, D, H, W), vq_loss)


# speedup vs baseline: 1.6415x; 1.0640x over previous
import jax
import jax.numpy as jnp
from jax.experimental import pallas as pl
from jax.experimental.pallas import tpu as pltpu

BETA_P1 = 1.25


def _vq_body(x_ref, cb_ref, cbt_ref, q_ref, loss_ref):
    b = pl.program_id(0)
    cb = cb_ref[...]
    C = cb_ref.shape[0]
    G = x_ref.shape[0]

    cb_sq = jnp.sum(cb * cb, axis=1, keepdims=True)
    iota_c = jax.lax.broadcasted_iota(jnp.int32, (C, 1), 0).astype(jnp.float32)

    @pl.when(b == 0)
    def _():
        loss_ref[0, 0] = 0.0

    mms = [
        jax.lax.dot_general(
            cb, x_ref[g], (((1,), (0,)), ((), ())),
            preferred_element_type=jnp.float32)
        for g in range(G)
    ]
    for g in range(G):
        xb = x_ref[g]
        x_sq = jnp.sum(xb * xb, axis=0, keepdims=True)
        mm = mms[g]
        dist = (0.5 * x_sq + 0.5 * cb_sq) - mm

        minv = jnp.min(dist, axis=0, keepdims=True)
        masked = jnp.where(dist == minv, iota_c, jnp.float32(C))
        idx = jnp.min(masked, axis=0, keepdims=True)
        onehot = (iota_c == idx).astype(jnp.float32)

        q = jax.lax.dot_general(
            cbt_ref[...], onehot, (((1,), (0,)), ((), ())),
            preferred_element_type=jnp.float32)
        q_ref[g] = q
        loss_ref[0, 0] += jnp.sum(minv)


def kernel(latents, codebook):
    B, D, H, W = latents.shape
    C = codebook.shape[0]
    HW = H * W
    x = latents.reshape(B, D, HW)
    cbt = codebook.T

    G = 4
    q, s = pl.pallas_call(
        _vq_body,
        grid=(B // G,),
        in_specs=[
            pl.BlockSpec((G, D, HW), lambda b: (b, 0, 0)),
            pl.BlockSpec((C, D), lambda b: (0, 0)),
            pl.BlockSpec((D, C), lambda b: (0, 0)),
        ],
        out_specs=[
            pl.BlockSpec((G, D, HW), lambda b: (b, 0, 0)),
            pl.BlockSpec(memory_space=pltpu.SMEM),
        ],
        out_shape=[
            jax.ShapeDtypeStruct((B, D, HW), jnp.float32),
            jax.ShapeDtypeStruct((1, 1), jnp.float32),
        ],
    )(x, codebook, cbt)

    vq_loss = (2.0 * BETA_P1 / (B * HW * D)) * s[0, 0]
    return (q.reshape(B, D, H, W), vq_loss)
